# baseline (speedup 1.0000x reference)
import jax
import jax.numpy as jnp
from jax.experimental import pallas as pl
from jax.experimental.pallas import tpu as pltpu

SEQ_BLOCK = 1024


def _body(t_ref, x_ref, p_ref, o_ref, pscr_ref, psem, xsem):
    s = pl.program_id(0)
    ns = pl.num_programs(0)
    B = pscr_ref.shape[0]

    def row_cp(b):
        return pltpu.make_async_copy(
            p_ref.at[pl.ds(t_ref[b], 1), :],
            pscr_ref.at[pl.ds(b, 1), :],
            psem)

    def x_cp(b):
        return pltpu.make_async_copy(
            x_ref.at[b, pl.ds(s * SEQ_BLOCK, SEQ_BLOCK), :],
            o_ref.at[:, b, :],
            xsem)

    @pl.when(s == 0)
    def _start_lookup():
        for b in range(B):
            row_cp(b).start()

    @pl.when(s < ns - 1)
    def _copy():
        for b in range(B):
            x_cp(b).start()
        for b in range(B):
            x_cp(b).wait()

    @pl.when(s == ns - 1)
    def _prompt_rows():
        for b in range(B):
            row_cp(b).wait()
        o_ref[0, :, :] = pscr_ref[...]


def kernel(x, task_id, prompt):
    B, S, D = x.shape
    n_sb = S // SEQ_BLOCK

    grid_spec = pltpu.PrefetchScalarGridSpec(
        num_scalar_prefetch=1,
        grid=(n_sb + 1,),
        in_specs=[
            pl.BlockSpec(memory_space=pl.ANY),
            pl.BlockSpec(memory_space=pl.ANY),
        ],
        out_specs=pl.BlockSpec((SEQ_BLOCK, B, D), lambda s, t: (s, 0, 0)),
        scratch_shapes=[
            pltpu.VMEM((B, D), jnp.float32),
            pltpu.SemaphoreType.DMA,
            pltpu.SemaphoreType.DMA,
        ],
    )
    out_t = pl.pallas_call(
        _body,
        grid_spec=grid_spec,
        out_shape=jax.ShapeDtypeStruct((S + 1, B, D), x.dtype),
    )(task_id, x, prompt)
    out = jnp.transpose(out_t, (1, 0, 2))
    return (out, task_id)
